# baseline (speedup 1.0000x reference)
import functools

import jax
import jax.numpy as jnp
from jax import lax
from jax.experimental import pallas as pl
from jax.experimental.pallas import tpu as pltpu
from jax.experimental.pallas import tpu_sc as plsc

_HIDDEN = 32
_GAMMA = 12.0
_BATCH = 16384

_INFO = plsc.get_sparse_core_info()
_NC = _INFO.num_cores
_NS = _INFO.num_subcores
_NW = _NC * _NS
_PER_W = _BATCH // _NW
_CHUNK = 128
_NCHUNK = _PER_W // _CHUNK
_UNROLL = 8


def _sc_body(hidx_hbm, ridx_hbm, tidx_hbm, ent_hbm, rel_hbm, val_hbm,
             out_hbm, hidx_v, ridx_v, tidx_v, h_v, r_v, t_v, o_v, sem):
    wid = lax.axis_index("s") * _NC + lax.axis_index("c")
    row0 = wid * _NCHUNK

    pltpu.sync_copy(hidx_hbm.at[pl.ds(row0, _NCHUNK)], hidx_v)
    pltpu.sync_copy(ridx_hbm.at[pl.ds(row0, _NCHUNK)], ridx_v)
    pltpu.sync_copy(tidx_hbm.at[pl.ds(row0, _NCHUNK)], tidx_v)

    descs = []
    for j in range(_NCHUNK):
        dst = pl.ds(j * _CHUNK, _CHUNK)
        descs.append(pltpu.async_copy(ent_hbm.at[hidx_v.at[j]], h_v.at[dst], sem))
        descs.append(pltpu.async_copy(rel_hbm.at[ridx_v.at[j]], r_v.at[dst], sem))
        descs.append(pltpu.async_copy(val_hbm.at[tidx_v.at[j]], t_v.at[dst], sem))
    for d in descs:
        d.wait()

    lanes = lax.iota(jnp.int32, 16)

    def score_rows(i, _):
        acc = jnp.zeros((16,), jnp.float32)
        for k in range(16):
            row = i * 16 + k
            lo = pl.ds(0, 16)
            hi = pl.ds(16, 16)
            a = jnp.abs(h_v[row, lo] + r_v[row, lo] - t_v[row, lo])
            b = jnp.abs(h_v[row, hi] + r_v[row, hi] - t_v[row, hi])
            s = jnp.sum(a + b)
            acc = jnp.where(lanes == k, s, acc)
        o_v[pl.ds(i * 16, 16)] = _GAMMA - acc
        return ()

    lax.fori_loop(0, _PER_W // 16, score_rows, ())

    pltpu.sync_copy(o_v, out_hbm.at[pl.ds(wid * _PER_W, _PER_W)])


@jax.jit
def _sc_score(hidx, ridx, tidx, ent, rel, val):
    mesh = plsc.VectorSubcoreMesh(core_axis_name="c", subcore_axis_name="s")
    f = functools.partial(
        pl.kernel,
        mesh=mesh,
        compiler_params=pltpu.CompilerParams(
            needs_layout_passes=False, use_tc_tiling_on_sc=False),
        out_type=jax.ShapeDtypeStruct((_BATCH,), jnp.float32),
        scratch_types=[
            pltpu.VMEM((_NCHUNK, _CHUNK), jnp.int32),
            pltpu.VMEM((_NCHUNK, _CHUNK), jnp.int32),
            pltpu.VMEM((_NCHUNK, _CHUNK), jnp.int32),
            pltpu.VMEM((_PER_W, _HIDDEN), jnp.float32),
            pltpu.VMEM((_PER_W, _HIDDEN), jnp.float32),
            pltpu.VMEM((_PER_W, _HIDDEN), jnp.float32),
            pltpu.VMEM((_PER_W,), jnp.float32),
            pltpu.SemaphoreType.DMA,
        ],
    )(_sc_body)
    return f(hidx, ridx, tidx, ent, rel, val)


def kernel(sample, entity_embedding, relation_embedding, value_embedding):
    idx = sample.astype(jnp.int32).T
    hidx = idx[0].reshape(_BATCH // _CHUNK, _CHUNK)
    ridx = idx[1].reshape(_BATCH // _CHUNK, _CHUNK)
    tidx = idx[2].reshape(_BATCH // _CHUNK, _CHUNK)
    score = _sc_score(hidx, ridx, tidx, entity_embedding,
                      relation_embedding, value_embedding)
    return score.reshape(_BATCH, 1)


# speedup vs baseline: 4.1638x; 4.1638x over previous
import functools

import jax
import jax.numpy as jnp
from jax import lax
from jax.experimental import pallas as pl
from jax.experimental.pallas import tpu as pltpu
from jax.experimental.pallas import tpu_sc as plsc

_HIDDEN = 32
_GAMMA = 12.0
_BATCH = 16384

_INFO = plsc.get_sparse_core_info()
_NC = _INFO.num_cores
_NS = _INFO.num_subcores
_NW = _NC * _NS
_PER_W = _BATCH // _NW
_IDXROWS = _PER_W // 128
_GW = 128
_RING = 4


def _sc_body(hidx_hbm, ridx_hbm, tidx_hbm, entT, relT, valT,
             out_hbm, hidx_v, ridx_v, tidx_v, h_v, r_v, t_v, o_v, sem):
    wid = lax.axis_index("s") * _NC + lax.axis_index("c")
    row0 = wid * _IDXROWS

    pltpu.sync_copy(hidx_hbm.at[pl.ds(row0, _IDXROWS)], hidx_v)
    pltpu.sync_copy(ridx_hbm.at[pl.ds(row0, _IDXROWS)], ridx_v)
    pltpu.sync_copy(tidx_hbm.at[pl.ds(row0, _IDXROWS)], tidx_v)

    lanes = lax.iota(jnp.int32, 16)
    dlo = lax.iota(jnp.int32, 16)

    def _issue(vh, vr, vt, k):
        slot = k % _RING
        dst = pl.ds(slot * _HIDDEN, _HIDDEN)
        eh, er, et = vh[k], vr[k], vt[k]
        bh = pl.multiple_of((eh // _GW) * _GW, _GW)
        br = pl.multiple_of((er // _GW) * _GW, _GW)
        bt = pl.multiple_of((et // _GW) * _GW, _GW)
        pltpu.async_copy(entT.at[:, pl.ds(bh, _GW)], h_v.at[dst], sem)
        pltpu.async_copy(relT.at[:, pl.ds(br, _GW)], r_v.at[dst], sem)
        pltpu.async_copy(valT.at[:, pl.ds(bt, _GW)], t_v.at[dst], sem)

    def step(g, _):
        blk = g // 8
        off = (g % 8) * 16
        base16 = pl.ds(off, 16)
        vh = hidx_v[blk, base16]
        vr = ridx_v[blk, base16]
        vt = tidx_v[blk, base16]
        for k in range(_RING - 1):
            _issue(vh, vr, vt, k)
        acc = jnp.zeros((16,), jnp.float32)
        for k in range(16):
            if k + _RING - 1 < 16:
                _issue(vh, vr, vt, k + _RING - 1)
            pltpu.make_async_copy(entT.at[:, pl.ds(0, _GW)],
                                  h_v.at[pl.ds(0, _HIDDEN)], sem).wait()
            pltpu.make_async_copy(relT.at[:, pl.ds(0, _GW)],
                                  r_v.at[pl.ds(0, _HIDDEN)], sem).wait()
            pltpu.make_async_copy(valT.at[:, pl.ds(0, _GW)],
                                  t_v.at[pl.ds(0, _HIDDEN)], sem).wait()
            slot = (k % _RING) * _HIDDEN
            eh, er, et = vh[k], vr[k], vt[k]
            lh = jnp.full((16,), eh % _GW, jnp.int32)
            lr = jnp.full((16,), er % _GW, jnp.int32)
            lt = jnp.full((16,), et % _GW, jnp.int32)
            h0 = plsc.load_gather(h_v, [slot + dlo, lh])
            h1 = plsc.load_gather(h_v, [slot + 16 + dlo, lh])
            r0 = plsc.load_gather(r_v, [slot + dlo, lr])
            r1 = plsc.load_gather(r_v, [slot + 16 + dlo, lr])
            t0 = plsc.load_gather(t_v, [slot + dlo, lt])
            t1 = plsc.load_gather(t_v, [slot + 16 + dlo, lt])
            part = jnp.abs(h0 + r0 - t0) + jnp.abs(h1 + r1 - t1)
            acc = jnp.where(lanes == k, _GAMMA - jnp.sum(part), acc)
        o_v[pl.ds(g * 16, 16)] = acc
        return ()

    lax.fori_loop(0, _PER_W // 16, step, ())

    pltpu.sync_copy(o_v, out_hbm.at[pl.ds(wid * _PER_W, _PER_W)])


@jax.jit
def _sc_score(hidx, ridx, tidx, entT, relT, valT):
    mesh = plsc.VectorSubcoreMesh(core_axis_name="c", subcore_axis_name="s")
    f = functools.partial(
        pl.kernel,
        mesh=mesh,
        compiler_params=pltpu.CompilerParams(needs_layout_passes=False),
        out_type=jax.ShapeDtypeStruct((_BATCH,), jnp.float32),
        scratch_types=[
            pltpu.VMEM((_IDXROWS, 128), jnp.int32),
            pltpu.VMEM((_IDXROWS, 128), jnp.int32),
            pltpu.VMEM((_IDXROWS, 128), jnp.int32),
            pltpu.VMEM((_RING * _HIDDEN, _GW), jnp.float32),
            pltpu.VMEM((_RING * _HIDDEN, _GW), jnp.float32),
            pltpu.VMEM((_RING * _HIDDEN, _GW), jnp.float32),
            pltpu.VMEM((_PER_W,), jnp.float32),
            pltpu.SemaphoreType.DMA,
        ],
    )(_sc_body)
    return f(hidx, ridx, tidx, entT, relT, valT)


def kernel(sample, entity_embedding, relation_embedding, value_embedding):
    idx = sample.astype(jnp.int32).T
    hidx = idx[0].reshape(_BATCH // 128, 128)
    ridx = idx[1].reshape(_BATCH // 128, 128)
    tidx = idx[2].reshape(_BATCH // 128, 128)
    score = _sc_score(hidx, ridx, tidx, entity_embedding.T,
                      relation_embedding.T, value_embedding.T)
    return score.reshape(_BATCH, 1)
